# baseline (speedup 1.0000x reference)
import functools

import jax
import jax.numpy as jnp
from jax import lax
from jax.experimental import pallas as pl
from jax.experimental.pallas import tpu as pltpu
from jax.experimental.pallas import tpu_sc as plsc

NUM_CORES = 2
NUM_SUBCORES = 16
NW = NUM_CORES * NUM_SUBCORES


def _sc_gather_pairs(table2, idx_pair, chunk):
    n = idx_pair.shape[0]
    d2 = table2.shape[1]
    per_w = n // NW
    n_chunks = per_w // chunk
    mesh = plsc.VectorSubcoreMesh(core_axis_name="c", subcore_axis_name="s")

    @functools.partial(
        pl.kernel,
        mesh=mesh,
        out_type=jax.ShapeDtypeStruct((n, d2), jnp.float32),
        scratch_types=[
            pltpu.VMEM((chunk,), jnp.int32),
            pltpu.VMEM((chunk, d2), jnp.float32),
            pltpu.SemaphoreType.DMA,
        ],
    )
    def gather_kernel(table_hbm, idx_hbm, out_hbm, idx_v, rows_v, sem):
        wid = lax.axis_index("s") * NUM_CORES + lax.axis_index("c")
        base = wid * per_w

        @pl.loop(0, n_chunks)
        def _(c):
            off = base + c * chunk
            pltpu.sync_copy(idx_hbm.at[pl.ds(off, chunk)], idx_v)
            pltpu.async_copy(table_hbm.at[idx_v], rows_v, sem).wait()
            pltpu.sync_copy(rows_v, out_hbm.at[pl.ds(off, chunk)])

    return gather_kernel(table2, idx_pair)


def _tc_select_project(emb2, parity, W, b, bm):
    n, d2 = emb2.shape
    d = d2 // 2
    o = W.shape[0]

    def mm_kernel(x_ref, p_ref, w_ref, b_ref, o_ref):
        lo = x_ref[:, :d]
        hi = x_ref[:, d:]
        x = jnp.where(p_ref[...] > 0, hi, lo)
        acc = jax.lax.dot_general(
            x, w_ref[...],
            dimension_numbers=(((1,), (1,)), ((), ())),
            preferred_element_type=jnp.float32,
        )
        o_ref[...] = acc + b_ref[...]

    return pl.pallas_call(
        mm_kernel,
        grid=(n // bm,),
        in_specs=[
            pl.BlockSpec((bm, d2), lambda i: (i, 0)),
            pl.BlockSpec((bm, 1), lambda i: (i, 0)),
            pl.BlockSpec((o, d), lambda i: (0, 0)),
            pl.BlockSpec((1, o), lambda i: (0, 0)),
        ],
        out_specs=pl.BlockSpec((bm, o), lambda i: (i, 0)),
        out_shape=jax.ShapeDtypeStruct((n, o), jnp.float32),
    )(emb2, parity, W, b.reshape(1, o))


def kernel(indexes, table, W, b):
    batch, hist = indexes.shape
    vocab, d = table.shape
    idx_flat = indexes.reshape(-1).astype(jnp.int32)
    idx_pair = idx_flat >> 1
    parity = (idx_flat & 1).reshape(-1, 1).astype(jnp.int32)
    table2 = table.reshape(vocab // 2, 2 * d)
    emb2 = _sc_gather_pairs(table2, idx_pair, chunk=512)
    out = _tc_select_project(emb2, parity, W, b, bm=4096)
    return out.reshape(batch, hist, W.shape[0])


# speedup vs baseline: 1.3749x; 1.3749x over previous
import functools

import jax
import jax.numpy as jnp
from jax import lax
from jax.experimental import pallas as pl
from jax.experimental.pallas import tpu as pltpu
from jax.experimental.pallas import tpu_sc as plsc

NUM_CORES = 2
NUM_SUBCORES = 16
NW = NUM_CORES * NUM_SUBCORES


def _tc_transpose(tT, bk):
    d, v = tT.shape

    def tr_kernel(x_ref, o_ref):
        o_ref[...] = x_ref[...].T

    return pl.pallas_call(
        tr_kernel,
        grid=(pl.cdiv(v, bk),),
        in_specs=[pl.BlockSpec((d, bk), lambda i: (0, i))],
        out_specs=pl.BlockSpec((bk, d), lambda i: (i, 0)),
        out_shape=jax.ShapeDtypeStruct((v, d), jnp.float32),
    )(tT)


def _sc_gather(table, idx_flat, chunk):
    n = idx_flat.shape[0]
    d = table.shape[1]
    per_w = n // NW
    n_chunks = per_w // chunk
    mesh = plsc.VectorSubcoreMesh(core_axis_name="c", subcore_axis_name="s")

    @functools.partial(
        pl.kernel,
        mesh=mesh,
        compiler_params=pltpu.CompilerParams(use_tc_tiling_on_sc=False),
        out_type=jax.ShapeDtypeStruct((n, d), jnp.float32),
        scratch_types=[
            pltpu.VMEM((chunk,), jnp.int32),
            pltpu.VMEM((chunk, d), jnp.float32),
            pltpu.SemaphoreType.DMA,
        ],
    )
    def gather_kernel(table_hbm, idx_hbm, out_hbm, idx_v, rows_v, sem):
        wid = lax.axis_index("s") * NUM_CORES + lax.axis_index("c")
        base = wid * per_w

        @pl.loop(0, n_chunks)
        def _(c):
            off = base + c * chunk
            pltpu.sync_copy(idx_hbm.at[pl.ds(off, chunk)], idx_v)
            pltpu.async_copy(table_hbm.at[idx_v], rows_v, sem).wait()
            pltpu.sync_copy(rows_v, out_hbm.at[pl.ds(off, chunk)])

    return gather_kernel(table, idx_flat)


def _tc_project(emb, W, b, bm):
    n, d = emb.shape
    o = W.shape[0]

    def mm_kernel(x_ref, w_ref, b_ref, o_ref):
        acc = jax.lax.dot_general(
            x_ref[...], w_ref[...],
            dimension_numbers=(((1,), (1,)), ((), ())),
            preferred_element_type=jnp.float32,
        )
        o_ref[...] = acc + b_ref[...]

    return pl.pallas_call(
        mm_kernel,
        grid=(n // bm,),
        in_specs=[
            pl.BlockSpec((bm, d), lambda i: (i, 0)),
            pl.BlockSpec((o, d), lambda i: (0, 0)),
            pl.BlockSpec((1, o), lambda i: (0, 0)),
        ],
        out_specs=pl.BlockSpec((bm, o), lambda i: (i, 0)),
        out_shape=jax.ShapeDtypeStruct((n, o), jnp.float32),
    )(emb, W, b.reshape(1, o))


def kernel(indexes, table, W, b):
    batch, hist = indexes.shape
    vocab, d = table.shape
    o = W.shape[0]
    idx_t = indexes.T.reshape(-1).astype(jnp.int32)
    table_rm = _tc_transpose(table.T, bk=16384)
    emb = _sc_gather(table_rm, idx_t, chunk=1024)
    out = _tc_project(emb, W, b, bm=8192)
    return out.reshape(hist, batch, o).transpose(1, 0, 2)


# speedup vs baseline: 3.2469x; 2.3615x over previous
import functools

import jax
import jax.numpy as jnp
from jax import lax
from jax.experimental import pallas as pl
from jax.experimental.pallas import tpu as pltpu
from jax.experimental.pallas import tpu_sc as plsc

NUM_CORES = 2
NUM_SUBCORES = 16
NW = NUM_CORES * NUM_SUBCORES


def _tc_project_table(tT, W, b, bk):
    d, v = tT.shape
    o = W.shape[0]

    def proj_kernel(x_ref, w_ref, b_ref, o_ref):
        acc = jax.lax.dot_general(
            x_ref[...], w_ref[...],
            dimension_numbers=(((0,), (1,)), ((), ())),
            preferred_element_type=jnp.float32,
        )
        o_ref[...] = acc + b_ref[...]

    return pl.pallas_call(
        proj_kernel,
        grid=(pl.cdiv(v, bk),),
        in_specs=[
            pl.BlockSpec((d, bk), lambda i: (0, i)),
            pl.BlockSpec((o, d), lambda i: (0, 0)),
            pl.BlockSpec((1, o), lambda i: (0, 0)),
        ],
        out_specs=pl.BlockSpec((bk, o), lambda i: (i, 0)),
        out_shape=jax.ShapeDtypeStruct((v, o), jnp.float32),
    )(tT, W, b.reshape(1, o))


def _sc_gather(proj, idx_flat, chunk):
    n = idx_flat.shape[0]
    o = proj.shape[1]
    per_w = n // NW
    n_chunks = per_w // chunk
    mesh = plsc.VectorSubcoreMesh(core_axis_name="c", subcore_axis_name="s")

    @functools.partial(
        pl.kernel,
        mesh=mesh,
        out_type=jax.ShapeDtypeStruct((n, o), jnp.float32),
        scratch_types=[
            pltpu.VMEM((chunk,), jnp.int32),
            pltpu.VMEM((chunk, o), jnp.float32),
            pltpu.SemaphoreType.DMA,
        ],
    )
    def gather_kernel(proj_hbm, idx_hbm, out_hbm, idx_v, rows_v, sem):
        wid = lax.axis_index("s") * NUM_CORES + lax.axis_index("c")
        base = wid * per_w

        @pl.loop(0, n_chunks)
        def _(c):
            off = base + c * chunk
            pltpu.sync_copy(idx_hbm.at[pl.ds(off, chunk)], idx_v)
            pltpu.async_copy(proj_hbm.at[idx_v], rows_v, sem).wait()
            pltpu.sync_copy(rows_v, out_hbm.at[pl.ds(off, chunk)])

    return gather_kernel(proj, idx_flat)


def kernel(indexes, table, W, b):
    batch, hist = indexes.shape
    o = W.shape[0]
    idx_t = indexes.T.reshape(-1).astype(jnp.int32)
    proj = _tc_project_table(table.T, W, b, bk=16384)
    out = _sc_gather(proj, idx_t, chunk=512)
    return out.reshape(hist, batch, o).transpose(1, 0, 2)
